# baseline (speedup 1.0000x reference)
import functools

import jax
import jax.numpy as jnp
from jax import lax
from jax.experimental import pallas as pl
from jax.experimental.pallas import tpu as pltpu
from jax.experimental.pallas import tpu_sc as plsc

N = 10000
E = 320000
NC = 2
NS = 16
NW = NC * NS
EREAL = E // NW
EPT = 10240
K = 40
CH = EPT // K
NPAD = 10240
RSUB = NPAD // NS
ZROWS = 128
BN = 2000


def _bcast_lane(v, j):
    idx = jnp.full((16, 1), j, jnp.int32)
    dnums = lax.GatherDimensionNumbers(
        offset_dims=(), collapsed_slice_dims=(0,), start_index_map=(0,))
    return lax.gather(v, idx, dnums, (1,),
                      mode=lax.GatherScatterMode.PROMISE_IN_BOUNDS)


def _prep_body(hin, W_ref, Ms_ref, Md_ref, htab_ref, dtab_ref, mx_ref, i):
    h = jnp.dot(hin, W_ref[...], preferred_element_type=jnp.float32)
    stab = jnp.dot(h, Ms_ref[...], preferred_element_type=jnp.float32)
    dtab = jnp.dot(h, Md_ref[...], preferred_element_type=jnp.float32)
    htab_ref[...] = jnp.concatenate([h, stab], axis=1)
    dtab_ref[...] = dtab
    m0 = jnp.max(stab, axis=0, keepdims=True)
    m1 = jnp.max(dtab, axis=0, keepdims=True)
    mx = jnp.concatenate(
        [m0, m1, jnp.full((6, 16), -jnp.inf, jnp.float32)], axis=0)

    @pl.when(i == 0)
    def _():
        mx_ref[...] = mx

    @pl.when(i > 0)
    def _():
        mx_ref[...] = jnp.maximum(mx_ref[...], mx)


def _tc0_kernel(x_ref, W_ref, Ms_ref, Md_ref, htab_ref, dtab_ref, mx_ref):
    _prep_body(x_ref[...], W_ref, Ms_ref, Md_ref, htab_ref, dtab_ref, mx_ref,
               pl.program_id(0))


def _combine(part_ref, b_ref, Rep_ref):
    acc = part_ref[0] + part_ref[1]
    num = acc[:, :128]
    den16 = acc[:, 128:144]
    denf = jnp.dot(den16, Rep_ref[...], preferred_element_type=jnp.float32)
    return num / (denf + 1e-30) + b_ref[...]


def _tc1_kernel(part_ref, b_ref, W_ref, Ms_ref, Md_ref, Rep_ref,
                htab_ref, dtab_ref, mx_ref):
    hin = _combine(part_ref, b_ref, Rep_ref)
    hin = jnp.where(hin > 0, hin, jnp.exp(hin) - 1.0)
    _prep_body(hin, W_ref, Ms_ref, Md_ref, htab_ref, dtab_ref, mx_ref,
               pl.program_id(0))


def _tc2_kernel(part_ref, b_ref, W2_ref, Ms_ref, Md_ref, Rep_ref,
                stab_ref, dtab_ref, mx_ref):
    hin = _combine(part_ref, b_ref, Rep_ref)
    z = jnp.dot(hin, W2_ref[...], preferred_element_type=jnp.float32)
    stab = jnp.dot(z, Ms_ref[...], preferred_element_type=jnp.float32)
    dtab = jnp.dot(z, Md_ref[...], preferred_element_type=jnp.float32)
    stab_ref[...] = stab
    dtab_ref[...] = dtab
    m0 = jnp.max(stab, axis=0, keepdims=True)
    m1 = jnp.max(dtab, axis=0, keepdims=True)
    mx = jnp.concatenate(
        [m0, m1, jnp.full((6, 16), -jnp.inf, jnp.float32)], axis=0)
    i = pl.program_id(0)

    @pl.when(i == 0)
    def _():
        mx_ref[...] = mx

    @pl.when(i > 0)
    def _():
        mx_ref[...] = jnp.maximum(mx_ref[...], mx)


def _tc3_kernel(part_ref, b2_ref, out_ref):
    acc = part_ref[0] + part_ref[1]
    num = acc[:, 1:2]
    den = acc[:, 0:1]
    bmean = jnp.sum(b2_ref[...]) * (1.0 / 40.0)
    out_ref[...] = num / (den + 1e-30) + bmean


def _tc_prep0(x, W, Ms, Md):
    grid = (N // BN,)
    return pl.pallas_call(
        _tc0_kernel,
        grid=grid,
        in_specs=[
            pl.BlockSpec((BN, 128), lambda i: (i, 0)),
            pl.BlockSpec((128, 128), lambda i: (0, 0)),
            pl.BlockSpec((128, 16), lambda i: (0, 0)),
            pl.BlockSpec((128, 16), lambda i: (0, 0)),
        ],
        out_specs=[
            pl.BlockSpec((BN, 144), lambda i: (i, 0)),
            pl.BlockSpec((BN, 16), lambda i: (i, 0)),
            pl.BlockSpec((8, 16), lambda i: (0, 0)),
        ],
        out_shape=[
            jax.ShapeDtypeStruct((N, 144), jnp.float32),
            jax.ShapeDtypeStruct((N, 16), jnp.float32),
            jax.ShapeDtypeStruct((8, 16), jnp.float32),
        ],
    )(x, W, Ms, Md)


def _tc_prep1(part, b, W, Ms, Md, Rep):
    grid = (N // BN,)
    return pl.pallas_call(
        _tc1_kernel,
        grid=grid,
        in_specs=[
            pl.BlockSpec((2, BN, 144), lambda i: (0, i, 0)),
            pl.BlockSpec((1, 128), lambda i: (0, 0)),
            pl.BlockSpec((128, 128), lambda i: (0, 0)),
            pl.BlockSpec((128, 16), lambda i: (0, 0)),
            pl.BlockSpec((128, 16), lambda i: (0, 0)),
            pl.BlockSpec((16, 128), lambda i: (0, 0)),
        ],
        out_specs=[
            pl.BlockSpec((BN, 144), lambda i: (i, 0)),
            pl.BlockSpec((BN, 16), lambda i: (i, 0)),
            pl.BlockSpec((8, 16), lambda i: (0, 0)),
        ],
        out_shape=[
            jax.ShapeDtypeStruct((N, 144), jnp.float32),
            jax.ShapeDtypeStruct((N, 16), jnp.float32),
            jax.ShapeDtypeStruct((8, 16), jnp.float32),
        ],
    )(part, b, W, Ms, Md, Rep)


def _tc_prep2(part, b, W2p, Ms, Md, Rep):
    grid = (N // BN,)
    return pl.pallas_call(
        _tc2_kernel,
        grid=grid,
        in_specs=[
            pl.BlockSpec((2, BN, 144), lambda i: (0, i, 0)),
            pl.BlockSpec((1, 128), lambda i: (0, 0)),
            pl.BlockSpec((128, 128), lambda i: (0, 0)),
            pl.BlockSpec((128, 16), lambda i: (0, 0)),
            pl.BlockSpec((128, 16), lambda i: (0, 0)),
            pl.BlockSpec((16, 128), lambda i: (0, 0)),
        ],
        out_specs=[
            pl.BlockSpec((BN, 16), lambda i: (i, 0)),
            pl.BlockSpec((BN, 16), lambda i: (i, 0)),
            pl.BlockSpec((8, 16), lambda i: (0, 0)),
        ],
        out_shape=[
            jax.ShapeDtypeStruct((N, 16), jnp.float32),
            jax.ShapeDtypeStruct((N, 16), jnp.float32),
            jax.ShapeDtypeStruct((8, 16), jnp.float32),
        ],
    )(part, b, W2p, Ms, Md, Rep)


def _tc_final(part2, b2):
    grid = (N // BN,)
    return pl.pallas_call(
        _tc3_kernel,
        grid=grid,
        in_specs=[
            pl.BlockSpec((2, BN, 16), lambda i: (0, i, 0)),
            pl.BlockSpec((1, 40), lambda i: (0, 0)),
        ],
        out_specs=[pl.BlockSpec((BN, 1), lambda i: (i, 0))],
        out_shape=[jax.ShapeDtypeStruct((N, 1), jnp.float32)],
    )(part2, b2)[0]


_MESH = plsc.VectorSubcoreMesh(core_axis_name="c", subcore_axis_name="s")
_SC_PARAMS = pltpu.CompilerParams(use_tc_tiling_on_sc=False)


def _sc_pipeline_body(W, compute):
    def body(tab1, tab2, mx, srcs, dsts, zrows, out, *scr):
        sidx = scr[0:8]
        didx = scr[8:16]
        b1 = scr[16:20]
        b2 = scr[20:24]
        mb = scr[24:26]
        mxv = scr[26]
        acc = scr[27]
        semi = scr[28:36]
        semg = scr[36:40]
        semsc = scr[40:42]
        cid = lax.axis_index("c")
        sid = lax.axis_index("s")
        base = (cid * NS + sid) * EPT

        pltpu.sync_copy(mx, mxv)
        creg = jnp.maximum(mxv[0, :] + mxv[1, :], 0.0)

        @pl.loop(0, RSUB // ZROWS)
        def _(t):
            pltpu.sync_copy(zrows,
                            acc.at[pl.ds(sid * RSUB + t * ZROWS, ZROWS)])

        plsc.subcore_barrier()

        def issue_idx(g, i):
            off = base + g * K
            pltpu.async_copy(srcs.at[pl.ds(off, K)], sidx[i], semi[i])
            pltpu.async_copy(dsts.at[pl.ds(off, K)], didx[i], semi[i])

        def wait_idx(i):
            pltpu.make_async_copy(srcs.at[pl.ds(base, K)], sidx[i],
                                  semi[i]).wait()
            pltpu.make_async_copy(dsts.at[pl.ds(base, K)], didx[i],
                                  semi[i]).wait()

        def issue_gather(b, i):
            pltpu.async_copy(tab1.at[sidx[i]], b1[b], semg[b])
            pltpu.async_copy(tab2.at[didx[i]], b2[b], semg[b])

        def wait_gather(b, i):
            pltpu.make_async_copy(tab1.at[sidx[i]], b1[b], semg[b]).wait()
            pltpu.make_async_copy(tab2.at[didx[i]], b2[b], semg[b]).wait()

        def issue_scatter(m, i):
            pltpu.async_copy(mb[m], acc.at[didx[i]], semsc[m], add=True)

        def wait_scatter(m, i):
            pltpu.make_async_copy(mb[m], acc.at[didx[i]], semsc[m]).wait()

        for j in range(5):
            issue_idx(j, j)
        for j in range(3):
            wait_idx(j)
            issue_gather(j, j)

        @pl.loop(0, CH // 8)
        def _(t):
            for s in range(8):
                g = 8 * t + s
                wait_gather(s % 4, s)
                if s >= 2:
                    wait_scatter(s % 2, s - 2)
                else:
                    @pl.when(t > 0)
                    def _():
                        wait_scatter(s % 2, (s - 2) % 8)

                @pl.when(g + 5 < CH)
                def _():
                    issue_idx(g + 5, (s + 5) % 8)

                compute(b1[s % 4], b2[s % 4], mb[s % 2], creg)
                issue_scatter(s % 2, s)

                @pl.when(g + 3 < CH)
                def _():
                    wait_idx((s + 3) % 8)
                    issue_gather((s + 3) % 4, (s + 3) % 8)

        wait_scatter(0, 6)
        wait_scatter(1, 7)

        plsc.subcore_barrier()

        @pl.loop(0, RSUB // ZROWS)
        def _(t):
            r0 = sid * RSUB + t * ZROWS
            pltpu.sync_copy(acc.at[pl.ds(r0, ZROWS)],
                            out.at[cid, pl.ds(r0, ZROWS)])

    return body


def _sc_scratch(W):
    return (
        [pltpu.VMEM((K,), jnp.int32)] * 8 +
        [pltpu.VMEM((K,), jnp.int32)] * 8 +
        [pltpu.VMEM((K, W), jnp.float32)] * 4 +
        [pltpu.VMEM((K, 16), jnp.float32)] * 4 +
        [pltpu.VMEM((K, W), jnp.float32)] * 2 +
        [pltpu.VMEM((8, 16), jnp.float32)] +
        [pltpu.VMEM_SHARED((NPAD, W), jnp.float32)] +
        [pltpu.SemaphoreType.DMA] * 14
    )


def _compute144(src_buf, dst_buf, msg_buf, creg):
    @plsc.parallel_loop(0, K, unroll=8)
    def _(k):
        a = dst_buf[k, :]
        r8 = src_buf[k, pl.ds(128, 16)]
        e = r8 + a
        e = jnp.maximum(e, 0.2 * e)
        w = jnp.exp(e - creg)
        msg_buf[k, pl.ds(128, 16)] = w
        for j in range(8):
            wj = _bcast_lane(w, j)
            msg_buf[k, pl.ds(16 * j, 16)] = src_buf[k, pl.ds(16 * j, 16)] * wj


def _compute16(src_buf, dst_buf, msg_buf, creg):
    lane = lax.iota(jnp.int32, 16)
    is0 = lane == 0
    is1 = lane == 1

    @plsc.parallel_loop(0, K, unroll=8)
    def _(k):
        g1 = src_buf[k, :]
        g2 = dst_buf[k, :]
        e = g1 + _bcast_lane(g2, 0)
        e = jnp.maximum(e, 0.2 * e)
        w = jnp.exp(e - creg)
        w0 = _bcast_lane(w, 0)
        prod = w0 * g1
        msg_buf[k, :] = jnp.where(is0, w0, jnp.where(is1, prod, 0.0))


_sc_edge144 = functools.partial(
    pl.kernel,
    mesh=_MESH,
    out_type=jax.ShapeDtypeStruct((2, NPAD, 144), jnp.float32),
    scratch_types=_sc_scratch(144),
    compiler_params=_SC_PARAMS,
)(_sc_pipeline_body(144, _compute144))


_sc_edge16 = functools.partial(
    pl.kernel,
    mesh=_MESH,
    out_type=jax.ShapeDtypeStruct((2, NPAD, 16), jnp.float32),
    scratch_types=_sc_scratch(16),
    compiler_params=_SC_PARAMS,
)(_sc_pipeline_body(16, _compute16))


def _build_M(a):
    H, C = a.shape
    rows = jnp.arange(H * C)
    col = jnp.arange(16)
    return jnp.where(col[None, :] == (rows // C)[:, None],
                     a.reshape(-1, 1), 0.0).astype(jnp.float32)


def kernel(x, edge_index, W0, as0, ad0, b0, W1, as1, ad1, b1,
           W2, as2, ad2, b2):
    ei = edge_index.astype(jnp.int32)
    padc = EPT - EREAL
    w = jnp.arange(NW, dtype=jnp.int32)[:, None]
    j = jnp.arange(padc, dtype=jnp.int32)[None, :]
    src_pad = (w * 977 + j * 37) % N
    dst_pad = N + (j + w * 7) % (NPAD - N)
    srcs = jnp.concatenate([ei[0].reshape(NW, EREAL), src_pad],
                           axis=1).reshape(-1)
    dsts = jnp.concatenate([ei[1].reshape(NW, EREAL), dst_pad],
                           axis=1).reshape(-1)

    Ms0, Md0 = _build_M(as0), _build_M(ad0)
    Ms1, Md1 = _build_M(as1), _build_M(ad1)

    W2p = jnp.pad(W2, ((0, 0), (0, 88)))
    col = jnp.arange(16)
    rows40 = jnp.arange(128)
    in40 = (rows40 < 40)[:, None]
    as2v = jnp.pad(as2.reshape(-1), (0, 88)).reshape(-1, 1)
    ad2v = jnp.pad(ad2.reshape(-1), (0, 88)).reshape(-1, 1)
    M2s = jnp.where((col[None, :] == 0) & in40, as2v, 0.0)
    M2s = jnp.where((col[None, :] == 1) & in40, 1.0 / 40.0, M2s)
    M2d = jnp.where((col[None, :] == 0) & in40, ad2v, 0.0)
    M2s = M2s.astype(jnp.float32)
    M2d = M2d.astype(jnp.float32)

    Rep = (jnp.arange(16)[:, None] == (jnp.arange(128)[None, :] // 16)
           ).astype(jnp.float32)

    b0r = b0.reshape(1, 128)
    b1r = b1.reshape(1, 128)
    b2r = b2.reshape(1, 40)

    z144 = jnp.zeros((ZROWS, 144), jnp.float32)
    z16 = jnp.zeros((ZROWS, 16), jnp.float32)

    htab0, dtab0, mx0 = _tc_prep0(x, W0, Ms0, Md0)
    p0 = _sc_edge144(htab0, dtab0, mx0, srcs, dsts, z144)
    htab1, dtab1, mx1 = _tc_prep1(p0, b0r, W1, Ms1, Md1, Rep)
    p1 = _sc_edge144(htab1, dtab1, mx1, srcs, dsts, z144)
    stab2, dtab2, mx2 = _tc_prep2(p1, b1r, W2p, M2s, M2d, Rep)
    p2 = _sc_edge16(stab2, dtab2, mx2, srcs, dsts, z16)
    out2d = _tc_final(p2, b2r)
    return out2d.reshape(N)


# speedup vs baseline: 1.0401x; 1.0401x over previous
import functools

import jax
import jax.numpy as jnp
from jax import lax
from jax.experimental import pallas as pl
from jax.experimental.pallas import tpu as pltpu
from jax.experimental.pallas import tpu_sc as plsc

N = 10000
E = 320000
NC = 2
NS = 16
NW = NC * NS
EREAL = E // NW
EPT = 10240
K = 40
CH = EPT // K
NPAD = 10240
RSUB = NPAD // NS
ZROWS = 128
BN = 2000


def _bcast_lane(v, j):
    idx = jnp.full((16, 1), j, jnp.int32)
    dnums = lax.GatherDimensionNumbers(
        offset_dims=(), collapsed_slice_dims=(0,), start_index_map=(0,))
    return lax.gather(v, idx, dnums, (1,),
                      mode=lax.GatherScatterMode.PROMISE_IN_BOUNDS)


def _prep_body(hin, W_ref, Ms_ref, Md_ref, htab_ref, stab_ref, dtab_ref,
               mx_ref, i):
    h = jnp.dot(hin, W_ref[...], preferred_element_type=jnp.float32)
    stab = jnp.dot(h, Ms_ref[...], preferred_element_type=jnp.float32)
    dtab = jnp.dot(h, Md_ref[...], preferred_element_type=jnp.float32)
    htab_ref[...] = h.astype(jnp.bfloat16)
    stab_ref[...] = stab
    dtab_ref[...] = dtab
    m0 = jnp.max(stab, axis=0, keepdims=True)
    m1 = jnp.max(dtab, axis=0, keepdims=True)
    mx = jnp.concatenate(
        [m0, m1, jnp.full((6, 16), -jnp.inf, jnp.float32)], axis=0)

    @pl.when(i == 0)
    def _():
        mx_ref[...] = mx

    @pl.when(i > 0)
    def _():
        mx_ref[...] = jnp.maximum(mx_ref[...], mx)


def _tc0_kernel(x_ref, W_ref, Ms_ref, Md_ref, htab_ref, stab_ref, dtab_ref,
                mx_ref):
    _prep_body(x_ref[...], W_ref, Ms_ref, Md_ref, htab_ref, stab_ref,
               dtab_ref, mx_ref, pl.program_id(0))


def _combine(part_ref, b_ref, Rep_ref):
    acc = part_ref[0] + part_ref[1]
    num = acc[:, :128]
    den16 = acc[:, 128:144]
    denf = jnp.dot(den16, Rep_ref[...], preferred_element_type=jnp.float32)
    return num / (denf + 1e-30) + b_ref[...]


def _tc1_kernel(part_ref, b_ref, W_ref, Ms_ref, Md_ref, Rep_ref,
                htab_ref, stab_ref, dtab_ref, mx_ref):
    hin = _combine(part_ref, b_ref, Rep_ref)
    hin = jnp.where(hin > 0, hin, jnp.exp(hin) - 1.0)
    _prep_body(hin, W_ref, Ms_ref, Md_ref, htab_ref, stab_ref, dtab_ref,
               mx_ref, pl.program_id(0))


def _tc2_kernel(part_ref, b_ref, W2_ref, Ms_ref, Md_ref, Rep_ref,
                stab_ref, dtab_ref, mx_ref):
    hin = _combine(part_ref, b_ref, Rep_ref)
    z = jnp.dot(hin, W2_ref[...], preferred_element_type=jnp.float32)
    stab = jnp.dot(z, Ms_ref[...], preferred_element_type=jnp.float32)
    dtab = jnp.dot(z, Md_ref[...], preferred_element_type=jnp.float32)
    stab_ref[...] = stab
    dtab_ref[...] = dtab
    m0 = jnp.max(stab, axis=0, keepdims=True)
    m1 = jnp.max(dtab, axis=0, keepdims=True)
    mx = jnp.concatenate(
        [m0, m1, jnp.full((6, 16), -jnp.inf, jnp.float32)], axis=0)
    i = pl.program_id(0)

    @pl.when(i == 0)
    def _():
        mx_ref[...] = mx

    @pl.when(i > 0)
    def _():
        mx_ref[...] = jnp.maximum(mx_ref[...], mx)


def _tc3_kernel(part_ref, b2_ref, out_ref):
    acc = part_ref[0] + part_ref[1]
    num = acc[:, 1:2]
    den = acc[:, 0:1]
    bmean = jnp.sum(b2_ref[...]) * (1.0 / 40.0)
    out_ref[...] = num / (den + 1e-30) + bmean


def _tc_prep0(x, W, Ms, Md):
    grid = (N // BN,)
    return pl.pallas_call(
        _tc0_kernel,
        grid=grid,
        in_specs=[
            pl.BlockSpec((BN, 128), lambda i: (i, 0)),
            pl.BlockSpec((128, 128), lambda i: (0, 0)),
            pl.BlockSpec((128, 16), lambda i: (0, 0)),
            pl.BlockSpec((128, 16), lambda i: (0, 0)),
        ],
        out_specs=[
            pl.BlockSpec((BN, 128), lambda i: (i, 0)),
            pl.BlockSpec((BN, 16), lambda i: (i, 0)),
            pl.BlockSpec((BN, 16), lambda i: (i, 0)),
            pl.BlockSpec((8, 16), lambda i: (0, 0)),
        ],
        out_shape=[
            jax.ShapeDtypeStruct((N, 128), jnp.bfloat16),
            jax.ShapeDtypeStruct((N, 16), jnp.float32),
            jax.ShapeDtypeStruct((N, 16), jnp.float32),
            jax.ShapeDtypeStruct((8, 16), jnp.float32),
        ],
    )(x, W, Ms, Md)


def _tc_prep1(part, b, W, Ms, Md, Rep):
    grid = (N // BN,)
    return pl.pallas_call(
        _tc1_kernel,
        grid=grid,
        in_specs=[
            pl.BlockSpec((2, BN, 144), lambda i: (0, i, 0)),
            pl.BlockSpec((1, 128), lambda i: (0, 0)),
            pl.BlockSpec((128, 128), lambda i: (0, 0)),
            pl.BlockSpec((128, 16), lambda i: (0, 0)),
            pl.BlockSpec((128, 16), lambda i: (0, 0)),
            pl.BlockSpec((16, 128), lambda i: (0, 0)),
        ],
        out_specs=[
            pl.BlockSpec((BN, 128), lambda i: (i, 0)),
            pl.BlockSpec((BN, 16), lambda i: (i, 0)),
            pl.BlockSpec((BN, 16), lambda i: (i, 0)),
            pl.BlockSpec((8, 16), lambda i: (0, 0)),
        ],
        out_shape=[
            jax.ShapeDtypeStruct((N, 128), jnp.bfloat16),
            jax.ShapeDtypeStruct((N, 16), jnp.float32),
            jax.ShapeDtypeStruct((N, 16), jnp.float32),
            jax.ShapeDtypeStruct((8, 16), jnp.float32),
        ],
    )(part, b, W, Ms, Md, Rep)


def _tc_prep2(part, b, W2p, Ms, Md, Rep):
    grid = (N // BN,)
    return pl.pallas_call(
        _tc2_kernel,
        grid=grid,
        in_specs=[
            pl.BlockSpec((2, BN, 144), lambda i: (0, i, 0)),
            pl.BlockSpec((1, 128), lambda i: (0, 0)),
            pl.BlockSpec((128, 128), lambda i: (0, 0)),
            pl.BlockSpec((128, 16), lambda i: (0, 0)),
            pl.BlockSpec((128, 16), lambda i: (0, 0)),
            pl.BlockSpec((16, 128), lambda i: (0, 0)),
        ],
        out_specs=[
            pl.BlockSpec((BN, 16), lambda i: (i, 0)),
            pl.BlockSpec((BN, 16), lambda i: (i, 0)),
            pl.BlockSpec((8, 16), lambda i: (0, 0)),
        ],
        out_shape=[
            jax.ShapeDtypeStruct((N, 16), jnp.float32),
            jax.ShapeDtypeStruct((N, 16), jnp.float32),
            jax.ShapeDtypeStruct((8, 16), jnp.float32),
        ],
    )(part, b, W2p, Ms, Md, Rep)


def _tc_final(part2, b2):
    grid = (N // BN,)
    return pl.pallas_call(
        _tc3_kernel,
        grid=grid,
        in_specs=[
            pl.BlockSpec((2, BN, 16), lambda i: (0, i, 0)),
            pl.BlockSpec((1, 40), lambda i: (0, 0)),
        ],
        out_specs=[pl.BlockSpec((BN, 1), lambda i: (i, 0))],
        out_shape=[jax.ShapeDtypeStruct((N, 1), jnp.float32)],
    )(part2, b2)[0]


_MESH = plsc.VectorSubcoreMesh(core_axis_name="c", subcore_axis_name="s")
_SC_PARAMS = pltpu.CompilerParams(use_tc_tiling_on_sc=False,
                                  needs_layout_passes=False)


def _sc_pipeline_body(W, compute):
    def body(tab1, tab2, mx, srcs, dsts, zrows, out, *scr):
        sidx = scr[0:8]
        didx = scr[8:16]
        b1 = scr[16:20]
        b2 = scr[20:24]
        mb = scr[24:26]
        mxv = scr[26]
        acc = scr[27]
        semi = scr[28:36]
        semg = scr[36:40]
        semsc = scr[40:42]
        cid = lax.axis_index("c")
        sid = lax.axis_index("s")
        base = (cid * NS + sid) * EPT

        pltpu.sync_copy(mx, mxv)
        creg = jnp.maximum(mxv[0, :] + mxv[1, :], 0.0)

        @pl.loop(0, RSUB // ZROWS)
        def _(t):
            pltpu.sync_copy(zrows,
                            acc.at[pl.ds(sid * RSUB + t * ZROWS, ZROWS)])

        plsc.subcore_barrier()

        def issue_idx(g, i):
            off = base + g * K
            pltpu.async_copy(srcs.at[pl.ds(off, K)], sidx[i], semi[i])
            pltpu.async_copy(dsts.at[pl.ds(off, K)], didx[i], semi[i])

        def wait_idx(i):
            pltpu.make_async_copy(srcs.at[pl.ds(base, K)], sidx[i],
                                  semi[i]).wait()
            pltpu.make_async_copy(dsts.at[pl.ds(base, K)], didx[i],
                                  semi[i]).wait()

        def issue_gather(b, i):
            pltpu.async_copy(tab1.at[sidx[i]], b1[b], semg[b])
            pltpu.async_copy(tab2.at[didx[i]], b2[b], semg[b])

        def wait_gather(b, i):
            pltpu.make_async_copy(tab1.at[sidx[i]], b1[b], semg[b]).wait()
            pltpu.make_async_copy(tab2.at[didx[i]], b2[b], semg[b]).wait()

        def issue_scatter(m, i):
            pltpu.async_copy(mb[m], acc.at[didx[i]], semsc[m], add=True)

        def wait_scatter(m, i):
            pltpu.make_async_copy(mb[m], acc.at[didx[i]], semsc[m]).wait()

        for j in range(5):
            issue_idx(j, j)
        for j in range(3):
            wait_idx(j)
            issue_gather(j, j)

        @pl.loop(0, CH // 8)
        def _(t):
            for s in range(8):
                g = 8 * t + s
                wait_gather(s % 4, s)
                if s >= 2:
                    wait_scatter(s % 2, s - 2)
                else:
                    @pl.when(t > 0)
                    def _():
                        wait_scatter(s % 2, (s - 2) % 8)

                @pl.when(g + 5 < CH)
                def _():
                    issue_idx(g + 5, (s + 5) % 8)

                compute(b1[s % 4], b2[s % 4], mb[s % 2], creg)
                issue_scatter(s % 2, s)

                @pl.when(g + 3 < CH)
                def _():
                    wait_idx((s + 3) % 8)
                    issue_gather((s + 3) % 4, (s + 3) % 8)

        wait_scatter(0, 6)
        wait_scatter(1, 7)

        plsc.subcore_barrier()

        @pl.loop(0, RSUB // ZROWS)
        def _(t):
            r0 = sid * RSUB + t * ZROWS
            pltpu.sync_copy(acc.at[pl.ds(r0, ZROWS)],
                            out.at[cid, pl.ds(r0, ZROWS)])

    return body


def _sc_scratch(W, gwidth, gdtype):
    return (
        [pltpu.VMEM((K,), jnp.int32)] * 8 +
        [pltpu.VMEM((K,), jnp.int32)] * 8 +
        [pltpu.VMEM((K, gwidth), gdtype)] * 4 +
        [pltpu.VMEM((K, 16), jnp.float32)] * 4 +
        [pltpu.VMEM((K, W), jnp.float32)] * 2 +
        [pltpu.VMEM((8, 16), jnp.float32)] +
        [pltpu.VMEM_SHARED((NPAD, W), jnp.float32)] +
        [pltpu.SemaphoreType.DMA] * 14
    )


def _compute144(src_buf, dst_buf, msg_buf, creg):
    @plsc.parallel_loop(0, K, unroll=8)
    def _(k):
        a = dst_buf[k, :]
        r8 = plsc.bitcast(src_buf[k, pl.ds(128, 32)], jnp.float32)
        e = r8 + a
        e = jnp.maximum(e, 0.2 * e)
        w = jnp.exp(e - creg)
        msg_buf[k, pl.ds(128, 16)] = w
        for j in range(4):
            hab = src_buf[k, pl.ds(32 * j, 32)]
            ha, hb = plsc.unpack(hab, format=plsc.PackFormat.INTERLEAVED,
                                 preferred_element_type=jnp.float32)
            msg_buf[k, pl.ds(32 * j, 16)] = ha * _bcast_lane(w, 2 * j)
            msg_buf[k, pl.ds(32 * j + 16, 16)] = hb * _bcast_lane(w, 2 * j + 1)


def _compute16(src_buf, dst_buf, msg_buf, creg):
    lane = lax.iota(jnp.int32, 16)
    is0 = lane == 0
    is1 = lane == 1

    @plsc.parallel_loop(0, K, unroll=8)
    def _(k):
        g1 = src_buf[k, :]
        g2 = dst_buf[k, :]
        e = g1 + _bcast_lane(g2, 0)
        e = jnp.maximum(e, 0.2 * e)
        w = jnp.exp(e - creg)
        w0 = _bcast_lane(w, 0)
        prod = w0 * g1
        msg_buf[k, :] = jnp.where(is0, w0, jnp.where(is1, prod, 0.0))


_sc_edge144 = functools.partial(
    pl.kernel,
    mesh=_MESH,
    out_type=jax.ShapeDtypeStruct((2, NPAD, 144), jnp.float32),
    scratch_types=_sc_scratch(144, 160, jnp.bfloat16),
    compiler_params=_SC_PARAMS,
)(_sc_pipeline_body(144, _compute144))


_sc_edge16 = functools.partial(
    pl.kernel,
    mesh=_MESH,
    out_type=jax.ShapeDtypeStruct((2, NPAD, 16), jnp.float32),
    scratch_types=_sc_scratch(16, 16, jnp.float32),
    compiler_params=_SC_PARAMS,
)(_sc_pipeline_body(16, _compute16))


def _build_M(a):
    H, C = a.shape
    rows = jnp.arange(H * C)
    col = jnp.arange(16)
    return jnp.where(col[None, :] == (rows // C)[:, None],
                     a.reshape(-1, 1), 0.0).astype(jnp.float32)


def kernel(x, edge_index, W0, as0, ad0, b0, W1, as1, ad1, b1,
           W2, as2, ad2, b2):
    ei = edge_index.astype(jnp.int32)
    padc = EPT - EREAL
    w = jnp.arange(NW, dtype=jnp.int32)[:, None]
    j = jnp.arange(padc, dtype=jnp.int32)[None, :]
    src_pad = (w * 977 + j * 37) % N
    dst_pad = N + (j + w * 7) % (NPAD - N)
    srcs = jnp.concatenate([ei[0].reshape(NW, EREAL), src_pad],
                           axis=1).reshape(-1)
    dsts = jnp.concatenate([ei[1].reshape(NW, EREAL), dst_pad],
                           axis=1).reshape(-1)

    perm = jnp.asarray([32 * b + (k % 2) * 16 + k // 2
                        for b in range(4) for k in range(32)], jnp.int32)
    W0p = W0[:, perm]
    W1p = W1[:, perm]
    Ms0, Md0 = _build_M(as0)[perm, :], _build_M(ad0)[perm, :]
    Ms1, Md1 = _build_M(as1)[perm, :], _build_M(ad1)[perm, :]

    W2p = jnp.pad(W2, ((0, 0), (0, 88)))
    col = jnp.arange(16)
    rows40 = jnp.arange(128)
    in40 = (rows40 < 40)[:, None]
    as2v = jnp.pad(as2.reshape(-1), (0, 88)).reshape(-1, 1)
    ad2v = jnp.pad(ad2.reshape(-1), (0, 88)).reshape(-1, 1)
    M2s = jnp.where((col[None, :] == 0) & in40, as2v, 0.0)
    M2s = jnp.where((col[None, :] == 1) & in40, 1.0 / 40.0, M2s)
    M2d = jnp.where((col[None, :] == 0) & in40, ad2v, 0.0)
    M2s = M2s.astype(jnp.float32)
    M2d = M2d.astype(jnp.float32)

    Rep = (jnp.arange(16)[:, None] == (jnp.arange(128)[None, :] // 16)
           ).astype(jnp.float32)

    b0r = b0.reshape(1, 128)
    b1r = b1.reshape(1, 128)
    b2r = b2.reshape(1, 40)

    z144 = jnp.zeros((ZROWS, 144), jnp.float32)
    z16 = jnp.zeros((ZROWS, 16), jnp.float32)

    def _assemble_htab(hb, stab):
        asrcb = lax.bitcast_convert_type(stab[:, :8],
                                         jnp.bfloat16).reshape(N, 16)
        return jnp.concatenate(
            [hb, asrcb, jnp.zeros((N, 16), jnp.bfloat16)], axis=1)

    hb0, stab0, dtab0, mx0 = _tc_prep0(x, W0p, Ms0, Md0)
    p0 = _sc_edge144(_assemble_htab(hb0, stab0), dtab0, mx0, srcs, dsts, z144)
    hb1, stab1, dtab1, mx1 = _tc_prep1(p0, b0r, W1p, Ms1, Md1, Rep)
    p1 = _sc_edge144(_assemble_htab(hb1, stab1), dtab1, mx1, srcs, dsts, z144)
    stab2, dtab2, mx2 = _tc_prep2(p1, b1r, W2p, M2s, M2d, Rep)
    p2 = _sc_edge16(stab2, dtab2, mx2, srcs, dsts, z16)
    out2d = _tc_final(p2, b2r)
    return out2d.reshape(N)


# speedup vs baseline: 1.0473x; 1.0069x over previous
import functools

import jax
import jax.numpy as jnp
from jax import lax
from jax.experimental import pallas as pl
from jax.experimental.pallas import tpu as pltpu
from jax.experimental.pallas import tpu_sc as plsc

N = 10000
E = 320000
NC = 2
NS = 16
NW = NC * NS
EREAL = E // NW
EPT = 10240
K = 40
CH = EPT // K
NPAD = 10240
RSUB = NPAD // NS
ZROWS = 128
BN = 2000


def _bcast_lane(v, j):
    idx = jnp.full((16, 1), j, jnp.int32)
    dnums = lax.GatherDimensionNumbers(
        offset_dims=(), collapsed_slice_dims=(0,), start_index_map=(0,))
    return lax.gather(v, idx, dnums, (1,),
                      mode=lax.GatherScatterMode.PROMISE_IN_BOUNDS)


def _prep_body(hin, W_ref, Ms_ref, Md_ref, htab_ref, stab_ref, dtab_ref,
               mx_ref, i):
    h = jnp.dot(hin, W_ref[...], preferred_element_type=jnp.float32)
    stab = jnp.dot(h, Ms_ref[...], preferred_element_type=jnp.float32)
    dtab = jnp.dot(h, Md_ref[...], preferred_element_type=jnp.float32)
    htab_ref[...] = h.astype(jnp.bfloat16)
    stab_ref[...] = stab
    dtab_ref[...] = dtab
    m0 = jnp.max(stab, axis=0, keepdims=True)
    m1 = jnp.max(dtab, axis=0, keepdims=True)
    mx = jnp.concatenate(
        [m0, m1, jnp.full((6, 16), -jnp.inf, jnp.float32)], axis=0)

    @pl.when(i == 0)
    def _():
        mx_ref[...] = mx

    @pl.when(i > 0)
    def _():
        mx_ref[...] = jnp.maximum(mx_ref[...], mx)


def _tc0_kernel(x_ref, W_ref, Ms_ref, Md_ref, htab_ref, stab_ref, dtab_ref,
                mx_ref):
    _prep_body(x_ref[...], W_ref, Ms_ref, Md_ref, htab_ref, stab_ref,
               dtab_ref, mx_ref, pl.program_id(0))


def _combine(part_ref, b_ref, Rep_ref):
    acc = part_ref[0] + part_ref[1]
    num = acc[:, :128]
    den16 = acc[:, 128:144]
    denf = jnp.dot(den16, Rep_ref[...], preferred_element_type=jnp.float32)
    return num / (denf + 1e-30) + b_ref[...]


def _tc1_kernel(part_ref, b_ref, W_ref, Ms_ref, Md_ref, Rep_ref,
                htab_ref, stab_ref, dtab_ref, mx_ref):
    hin = _combine(part_ref, b_ref, Rep_ref)
    hin = jnp.where(hin > 0, hin, jnp.exp(hin) - 1.0)
    _prep_body(hin, W_ref, Ms_ref, Md_ref, htab_ref, stab_ref, dtab_ref,
               mx_ref, pl.program_id(0))


def _tc2_kernel(part_ref, b_ref, W2_ref, Ms_ref, Md_ref, Rep_ref,
                stab_ref, dtab_ref, mx_ref):
    hin = _combine(part_ref, b_ref, Rep_ref)
    z = jnp.dot(hin, W2_ref[...], preferred_element_type=jnp.float32)
    stab = jnp.dot(z, Ms_ref[...], preferred_element_type=jnp.float32)
    dtab = jnp.dot(z, Md_ref[...], preferred_element_type=jnp.float32)
    stab_ref[...] = stab
    dtab_ref[...] = dtab
    m0 = jnp.max(stab, axis=0, keepdims=True)
    m1 = jnp.max(dtab, axis=0, keepdims=True)
    mx = jnp.concatenate(
        [m0, m1, jnp.full((6, 16), -jnp.inf, jnp.float32)], axis=0)
    i = pl.program_id(0)

    @pl.when(i == 0)
    def _():
        mx_ref[...] = mx

    @pl.when(i > 0)
    def _():
        mx_ref[...] = jnp.maximum(mx_ref[...], mx)


def _tc3_kernel(part_ref, b2_ref, out_ref):
    acc = part_ref[0] + part_ref[1]
    num = acc[:, 1:2]
    den = acc[:, 0:1]
    bmean = jnp.sum(b2_ref[...]) * (1.0 / 40.0)
    out_ref[...] = num / (den + 1e-30) + bmean


def _tc_prep0(x, W, Ms, Md):
    grid = (N // BN,)
    return pl.pallas_call(
        _tc0_kernel,
        grid=grid,
        in_specs=[
            pl.BlockSpec((BN, 128), lambda i: (i, 0)),
            pl.BlockSpec((128, 128), lambda i: (0, 0)),
            pl.BlockSpec((128, 16), lambda i: (0, 0)),
            pl.BlockSpec((128, 16), lambda i: (0, 0)),
        ],
        out_specs=[
            pl.BlockSpec((BN, 128), lambda i: (i, 0)),
            pl.BlockSpec((BN, 16), lambda i: (i, 0)),
            pl.BlockSpec((BN, 16), lambda i: (i, 0)),
            pl.BlockSpec((8, 16), lambda i: (0, 0)),
        ],
        out_shape=[
            jax.ShapeDtypeStruct((N, 128), jnp.bfloat16),
            jax.ShapeDtypeStruct((N, 16), jnp.float32),
            jax.ShapeDtypeStruct((N, 16), jnp.float32),
            jax.ShapeDtypeStruct((8, 16), jnp.float32),
        ],
    )(x, W, Ms, Md)


def _tc_prep1(part, b, W, Ms, Md, Rep):
    grid = (N // BN,)
    return pl.pallas_call(
        _tc1_kernel,
        grid=grid,
        in_specs=[
            pl.BlockSpec((2, BN, 144), lambda i: (0, i, 0)),
            pl.BlockSpec((1, 128), lambda i: (0, 0)),
            pl.BlockSpec((128, 128), lambda i: (0, 0)),
            pl.BlockSpec((128, 16), lambda i: (0, 0)),
            pl.BlockSpec((128, 16), lambda i: (0, 0)),
            pl.BlockSpec((16, 128), lambda i: (0, 0)),
        ],
        out_specs=[
            pl.BlockSpec((BN, 128), lambda i: (i, 0)),
            pl.BlockSpec((BN, 16), lambda i: (i, 0)),
            pl.BlockSpec((BN, 16), lambda i: (i, 0)),
            pl.BlockSpec((8, 16), lambda i: (0, 0)),
        ],
        out_shape=[
            jax.ShapeDtypeStruct((N, 128), jnp.bfloat16),
            jax.ShapeDtypeStruct((N, 16), jnp.float32),
            jax.ShapeDtypeStruct((N, 16), jnp.float32),
            jax.ShapeDtypeStruct((8, 16), jnp.float32),
        ],
    )(part, b, W, Ms, Md, Rep)


def _tc_prep2(part, b, W2p, Ms, Md, Rep):
    grid = (N // BN,)
    return pl.pallas_call(
        _tc2_kernel,
        grid=grid,
        in_specs=[
            pl.BlockSpec((2, BN, 144), lambda i: (0, i, 0)),
            pl.BlockSpec((1, 128), lambda i: (0, 0)),
            pl.BlockSpec((128, 128), lambda i: (0, 0)),
            pl.BlockSpec((128, 16), lambda i: (0, 0)),
            pl.BlockSpec((128, 16), lambda i: (0, 0)),
            pl.BlockSpec((16, 128), lambda i: (0, 0)),
        ],
        out_specs=[
            pl.BlockSpec((BN, 16), lambda i: (i, 0)),
            pl.BlockSpec((BN, 16), lambda i: (i, 0)),
            pl.BlockSpec((8, 16), lambda i: (0, 0)),
        ],
        out_shape=[
            jax.ShapeDtypeStruct((N, 16), jnp.float32),
            jax.ShapeDtypeStruct((N, 16), jnp.float32),
            jax.ShapeDtypeStruct((8, 16), jnp.float32),
        ],
    )(part, b, W2p, Ms, Md, Rep)


def _tc_final(part2, b2):
    grid = (N // BN,)
    return pl.pallas_call(
        _tc3_kernel,
        grid=grid,
        in_specs=[
            pl.BlockSpec((2, BN, 16), lambda i: (0, i, 0)),
            pl.BlockSpec((1, 40), lambda i: (0, 0)),
        ],
        out_specs=[pl.BlockSpec((BN, 1), lambda i: (i, 0))],
        out_shape=[jax.ShapeDtypeStruct((N, 1), jnp.float32)],
    )(part2, b2)[0]


_MESH = plsc.VectorSubcoreMesh(core_axis_name="c", subcore_axis_name="s")
_SC_PARAMS = pltpu.CompilerParams(use_tc_tiling_on_sc=False,
                                  needs_layout_passes=False)


def _sc_pipeline_body(W, compute):
    def body(tab1, tab2, mx, srcs, dsts, zrows, out, *scr):
        sidx = scr[0:8]
        didx = scr[8:16]
        b1 = scr[16:20]
        b2 = scr[20:24]
        mb = scr[24:26]
        mxv = scr[26]
        acc = scr[27]
        semi = scr[28:36]
        semg = scr[36:40]
        semsc = scr[40:42]
        semz = scr[42]
        cid = lax.axis_index("c")
        sid = lax.axis_index("s")
        base = (cid * NS + sid) * EPT

        pltpu.sync_copy(mx, mxv)
        creg = jnp.maximum(mxv[0, :] + mxv[1, :], 0.0)

        def issue_idx(g, i):
            off = base + g * K
            pltpu.async_copy(srcs.at[pl.ds(off, K)], sidx[i], semi[i])
            pltpu.async_copy(dsts.at[pl.ds(off, K)], didx[i], semi[i])

        def wait_idx(i):
            pltpu.make_async_copy(srcs.at[pl.ds(base, K)], sidx[i],
                                  semi[i]).wait()
            pltpu.make_async_copy(dsts.at[pl.ds(base, K)], didx[i],
                                  semi[i]).wait()

        def issue_gather(b, i):
            pltpu.async_copy(tab1.at[sidx[i]], b1[b], semg[b])
            pltpu.async_copy(tab2.at[didx[i]], b2[b], semg[b])

        def wait_gather(b, i):
            pltpu.make_async_copy(tab1.at[sidx[i]], b1[b], semg[b]).wait()
            pltpu.make_async_copy(tab2.at[didx[i]], b2[b], semg[b]).wait()

        def issue_scatter(m, i):
            pltpu.async_copy(mb[m], acc.at[didx[i]], semsc[m], add=True)

        def wait_scatter(m, i):
            pltpu.make_async_copy(mb[m], acc.at[didx[i]], semsc[m]).wait()

        for j in range(5):
            issue_idx(j, j)
        for t in range(RSUB // ZROWS):
            pltpu.async_copy(
                zrows, acc.at[pl.ds(sid * RSUB + t * ZROWS, ZROWS)], semz)
        for j in range(3):
            wait_idx(j)
            issue_gather(j, j)
        for t in range(RSUB // ZROWS):
            pltpu.make_async_copy(
                zrows, acc.at[pl.ds(sid * RSUB + t * ZROWS, ZROWS)],
                semz).wait()

        plsc.subcore_barrier()

        @pl.loop(0, CH // 8)
        def _(t):
            for s in range(8):
                g = 8 * t + s
                wait_gather(s % 4, s)
                if s >= 2:
                    wait_scatter(s % 2, s - 2)
                else:
                    @pl.when(t > 0)
                    def _():
                        wait_scatter(s % 2, (s - 2) % 8)

                @pl.when(g + 5 < CH)
                def _():
                    issue_idx(g + 5, (s + 5) % 8)

                compute(b1[s % 4], b2[s % 4], mb[s % 2], creg)
                issue_scatter(s % 2, s)

                @pl.when(g + 3 < CH)
                def _():
                    wait_idx((s + 3) % 8)
                    issue_gather((s + 3) % 4, (s + 3) % 8)

        wait_scatter(0, 6)
        wait_scatter(1, 7)

        plsc.subcore_barrier()

        @pl.loop(0, RSUB // ZROWS)
        def _(t):
            r0 = sid * RSUB + t * ZROWS
            pltpu.sync_copy(acc.at[pl.ds(r0, ZROWS)],
                            out.at[cid, pl.ds(r0, ZROWS)])

    return body


def _sc_scratch(W, gwidth, gdtype):
    return (
        [pltpu.VMEM((K,), jnp.int32)] * 8 +
        [pltpu.VMEM((K,), jnp.int32)] * 8 +
        [pltpu.VMEM((K, gwidth), gdtype)] * 4 +
        [pltpu.VMEM((K, 16), jnp.float32)] * 4 +
        [pltpu.VMEM((K, W), jnp.float32)] * 2 +
        [pltpu.VMEM((8, 16), jnp.float32)] +
        [pltpu.VMEM_SHARED((NPAD, W), jnp.float32)] +
        [pltpu.SemaphoreType.DMA] * 15
    )


def _compute144(src_buf, dst_buf, msg_buf, creg):
    @plsc.parallel_loop(0, K, unroll=8)
    def _(k):
        a = dst_buf[k, :]
        r8 = plsc.bitcast(src_buf[k, pl.ds(128, 32)], jnp.float32)
        e = r8 + a
        e = jnp.maximum(e, 0.2 * e)
        w = jnp.exp(e - creg)
        msg_buf[k, pl.ds(128, 16)] = w
        for j in range(4):
            hab = src_buf[k, pl.ds(32 * j, 32)]
            ha, hb = plsc.unpack(hab, format=plsc.PackFormat.INTERLEAVED,
                                 preferred_element_type=jnp.float32)
            msg_buf[k, pl.ds(32 * j, 16)] = ha * _bcast_lane(w, 2 * j)
            msg_buf[k, pl.ds(32 * j + 16, 16)] = hb * _bcast_lane(w, 2 * j + 1)


def _compute16(src_buf, dst_buf, msg_buf, creg):
    lane = lax.iota(jnp.int32, 16)
    is0 = lane == 0
    is1 = lane == 1

    @plsc.parallel_loop(0, K, unroll=8)
    def _(k):
        g1 = src_buf[k, :]
        g2 = dst_buf[k, :]
        e = g1 + _bcast_lane(g2, 0)
        e = jnp.maximum(e, 0.2 * e)
        w = jnp.exp(e - creg)
        w0 = _bcast_lane(w, 0)
        prod = w0 * g1
        msg_buf[k, :] = jnp.where(is0, w0, jnp.where(is1, prod, 0.0))


_sc_edge144 = functools.partial(
    pl.kernel,
    mesh=_MESH,
    out_type=jax.ShapeDtypeStruct((2, NPAD, 144), jnp.float32),
    scratch_types=_sc_scratch(144, 160, jnp.bfloat16),
    compiler_params=_SC_PARAMS,
)(_sc_pipeline_body(144, _compute144))


_sc_edge16 = functools.partial(
    pl.kernel,
    mesh=_MESH,
    out_type=jax.ShapeDtypeStruct((2, NPAD, 16), jnp.float32),
    scratch_types=_sc_scratch(16, 16, jnp.float32),
    compiler_params=_SC_PARAMS,
)(_sc_pipeline_body(16, _compute16))


def _build_M(a):
    H, C = a.shape
    rows = jnp.arange(H * C)
    col = jnp.arange(16)
    return jnp.where(col[None, :] == (rows // C)[:, None],
                     a.reshape(-1, 1), 0.0).astype(jnp.float32)


def kernel(x, edge_index, W0, as0, ad0, b0, W1, as1, ad1, b1,
           W2, as2, ad2, b2):
    ei = edge_index.astype(jnp.int32)
    padc = EPT - EREAL
    w = jnp.arange(NW, dtype=jnp.int32)[:, None]
    j = jnp.arange(padc, dtype=jnp.int32)[None, :]
    src_pad = (w * 977 + j * 37) % N
    dst_pad = N + (j + w * 7) % (NPAD - N)
    srcs = jnp.concatenate([ei[0].reshape(NW, EREAL), src_pad],
                           axis=1).reshape(-1)
    dsts = jnp.concatenate([ei[1].reshape(NW, EREAL), dst_pad],
                           axis=1).reshape(-1)

    perm = jnp.asarray([32 * b + (k % 2) * 16 + k // 2
                        for b in range(4) for k in range(32)], jnp.int32)
    W0p = W0[:, perm]
    W1p = W1[:, perm]
    Ms0, Md0 = _build_M(as0)[perm, :], _build_M(ad0)[perm, :]
    Ms1, Md1 = _build_M(as1)[perm, :], _build_M(ad1)[perm, :]

    W2p = jnp.pad(W2, ((0, 0), (0, 88)))
    col = jnp.arange(16)
    rows40 = jnp.arange(128)
    in40 = (rows40 < 40)[:, None]
    as2v = jnp.pad(as2.reshape(-1), (0, 88)).reshape(-1, 1)
    ad2v = jnp.pad(ad2.reshape(-1), (0, 88)).reshape(-1, 1)
    M2s = jnp.where((col[None, :] == 0) & in40, as2v, 0.0)
    M2s = jnp.where((col[None, :] == 1) & in40, 1.0 / 40.0, M2s)
    M2d = jnp.where((col[None, :] == 0) & in40, ad2v, 0.0)
    M2s = M2s.astype(jnp.float32)
    M2d = M2d.astype(jnp.float32)

    Rep = (jnp.arange(16)[:, None] == (jnp.arange(128)[None, :] // 16)
           ).astype(jnp.float32)

    b0r = b0.reshape(1, 128)
    b1r = b1.reshape(1, 128)
    b2r = b2.reshape(1, 40)

    z144 = jnp.zeros((ZROWS, 144), jnp.float32)
    z16 = jnp.zeros((ZROWS, 16), jnp.float32)

    def _assemble_htab(hb, stab):
        asrcb = lax.bitcast_convert_type(stab[:, :8],
                                         jnp.bfloat16).reshape(N, 16)
        return jnp.concatenate(
            [hb, asrcb, jnp.zeros((N, 16), jnp.bfloat16)], axis=1)

    hb0, stab0, dtab0, mx0 = _tc_prep0(x, W0p, Ms0, Md0)
    p0 = _sc_edge144(_assemble_htab(hb0, stab0), dtab0, mx0, srcs, dsts, z144)
    hb1, stab1, dtab1, mx1 = _tc_prep1(p0, b0r, W1p, Ms1, Md1, Rep)
    p1 = _sc_edge144(_assemble_htab(hb1, stab1), dtab1, mx1, srcs, dsts, z144)
    stab2, dtab2, mx2 = _tc_prep2(p1, b1r, W2p, M2s, M2d, Rep)
    p2 = _sc_edge16(stab2, dtab2, mx2, srcs, dsts, z16)
    out2d = _tc_final(p2, b2r)
    return out2d.reshape(N)
